# Initial kernel scaffold; baseline (speedup 1.0000x reference)
import jax
import jax.numpy as jnp
from jax.experimental import pallas as pl


def kernel(x, w1, b1, s1, t1, a1, w2, b2, s2, t2, a2, w3, b3, s3, t3, a3):
    raise NotImplementedError("write your pallas kernel here")


# speedup vs baseline: 1.9975x; 1.9975x over previous
import jax
import jax.numpy as jnp
from jax.experimental import pallas as pl
from jax.experimental.pallas import tpu as pltpu


def _round_up(x, m):
    return (x + m - 1) // m * m


def _make_fused_kernel(H, W, L, Cout_p):
    Hp, Wp = H + 2, W + 2
    plane = Hp * Wp

    def _kernel_body(x_ref, w1_ref, w2_ref, w3_ref, alpha_ref, maskrow_ref,
                     o_ref, act_ref, patch_ref, b_ref):
        mask1 = maskrow_ref[0:1, :]

        def conv_bn_prelu(src, w_ref, li, pb):
            C = src.shape[0]
            patch = patch_ref.at[pb]
            b = b_ref.at[pb]
            for t in range(9):
                kh, kw = divmod(t, 3)
                s = (kh - 1) * Wp + (kw - 1)
                sh = (-s) % L
                piece = src if sh == 0 else pltpu.roll(src, sh, 1)
                patch[t * C:(t + 1) * C, :] = piece
            patch[9 * C:9 * C + 8, :] = maskrow_ref[...]
            b[...] = jnp.dot(w_ref[...], patch[:9 * C + 8, :],
                             preferred_element_type=jnp.float32
                             ).astype(jnp.bfloat16)
            acc = (b[Cout_p:2 * Cout_p, :]
                   + pltpu.roll(b[0:Cout_p, :], plane, 1)
                   + pltpu.roll(b[2 * Cout_p:3 * Cout_p, :], L - plane, 1))
            alpha = alpha_ref[li]
            y = jnp.where(acc > 0, acc, alpha * acc)
            return y * mask1

        for pb in range(2):
            act_ref[pb] = conv_bn_prelu(x_ref[pb], w1_ref, 0, pb)
        for pb in range(2):
            act_ref[pb] = conv_bn_prelu(act_ref[pb], w2_ref, 1, pb)
        for pb in range(2):
            o_ref[pb] = conv_bn_prelu(act_ref[pb], w3_ref, 2, pb)

    return _kernel_body


def _pack_weight(w, b, scale, shift, cin_p, cout_p):
    cin, cout = w.shape[3], w.shape[4]
    w_f = w * scale[None, None, None, None, :]
    w_p = jnp.pad(w_f, ((0, 0), (0, 0), (0, 0),
                        (0, cin_p - cin), (0, cout_p - cout)))
    w_t = jnp.transpose(w_p, (0, 4, 1, 2, 3))
    w_m = w_t.reshape(3 * cout_p, 9 * cin_p)
    shift_p = jnp.zeros((3 * cout_p,), jnp.float32).at[cout_p:cout_p + cout].set(
        shift + b * scale)
    extra = jnp.concatenate(
        [shift_p[:, None], jnp.zeros((3 * cout_p, 7), jnp.float32)], axis=1)
    return jnp.concatenate([w_m, extra], axis=1).astype(jnp.bfloat16)


def kernel(x, w1, b1, s1, t1, a1, w2, b2, s2, t2, a2, w3, b3, s3, t3, a3):
    N, Cin, D, H, W = x.shape
    Cout = w1.shape[-1]
    Dp, Hp, Wp = D + 2, H + 2, W + 2
    Ls = Dp * Hp * Wp
    L = _round_up(Ls, 128)
    Cin_p = _round_up(Cin, 8)
    Cout_p = _round_up(Cout, 8)

    xp = jnp.pad(x.astype(jnp.float32),
                 ((0, 0), (0, Cin_p - Cin), (1, 1), (1, 1), (1, 1)))
    x_flat = jnp.pad(xp.reshape(N, Cin_p, Ls),
                     ((0, 0), (0, 0), (0, L - Ls))).astype(jnp.bfloat16)

    interior = jnp.zeros((Dp, Hp, Wp), jnp.float32)
    interior = interior.at[1:1 + D, 1:1 + H, 1:1 + W].set(1.0)
    mask = jnp.pad(interior.reshape(1, Ls), ((0, 0), (0, L - Ls)))
    maskrow = jnp.concatenate(
        [mask, jnp.zeros((7, L), jnp.float32)], axis=0).astype(jnp.bfloat16)

    wf1 = _pack_weight(w1, b1, s1, t1, Cin_p, Cout_p)
    wf2 = _pack_weight(w2, b2, s2, t2, Cout_p, Cout_p)
    wf3 = _pack_weight(w3, b3, s3, t3, Cout_p, Cout_p)
    alphas = jnp.stack([
        jnp.broadcast_to(jnp.asarray(a, jnp.float32), (Cout_p,))
        for a in (a1, a2, a3)], axis=0).reshape(3, Cout_p, 1).astype(jnp.bfloat16)

    _fused = _make_fused_kernel(H, W, L, Cout_p)
    cmax = max(Cin_p, Cout_p)

    out_flat = pl.pallas_call(
        _fused,
        out_shape=jax.ShapeDtypeStruct((N, Cout_p, L), jnp.bfloat16),
        grid=(N // 2,),
        in_specs=[
            pl.BlockSpec((2, Cin_p, L), lambda n: (n, 0, 0)),
            pl.BlockSpec((3 * Cout_p, 9 * Cin_p + 8), lambda n: (0, 0)),
            pl.BlockSpec((3 * Cout_p, 9 * Cout_p + 8), lambda n: (0, 0)),
            pl.BlockSpec((3 * Cout_p, 9 * Cout_p + 8), lambda n: (0, 0)),
            pl.BlockSpec((3, Cout_p, 1), lambda n: (0, 0, 0)),
            pl.BlockSpec((8, L), lambda n: (0, 0)),
        ],
        out_specs=pl.BlockSpec((2, Cout_p, L), lambda n: (n, 0, 0)),
        scratch_shapes=[
            pltpu.VMEM((2, Cout_p, L), jnp.bfloat16),
            pltpu.VMEM((2, 9 * cmax + 8, L), jnp.bfloat16),
            pltpu.VMEM((2, 3 * Cout_p, L), jnp.bfloat16),
        ],
        compiler_params=pltpu.CompilerParams(
            dimension_semantics=("parallel",)),
    )(x_flat, wf1, wf2, wf3, alphas, maskrow)

    out = out_flat[:, :Cout, :Ls].reshape(N, Cout, Dp, Hp, Wp)
    return out[:, :, 1:1 + D, 1:1 + H, 1:1 + W].astype(jnp.float32)


# speedup vs baseline: 3.8169x; 1.9108x over previous
import jax
import jax.numpy as jnp
from jax.experimental import pallas as pl
from jax.experimental.pallas import tpu as pltpu


def _make_fused_kernel(D, H, W, Cout):
    V = D * H * W
    plane = H * W

    def _kernel_body(x_ref, w1_ref, w2_ref, w3_ref, alpha_ref, mask_ref,
                     o_ref, src_ref, act_ref, patch_ref, b_ref):

        def conv_bn_prelu(src, w_ref, li, pb):
            C = src.shape[0]
            patch = patch_ref.at[pb]
            b = b_ref.at[pb]
            for t in range(9):
                kh, kw = divmod(t, 3)
                s = (kh - 1) * W + (kw - 1)
                sh = (-s) % V
                piece = src if sh == 0 else pltpu.roll(src, sh, 1)
                patch[t * C:(t + 1) * C, :] = piece * mask_ref[t:t + 1, :]
            patch[9 * C:9 * C + 8, :] = mask_ref[9:17, :]
            b[...] = jnp.dot(w_ref[...], patch[:9 * C + 8, :],
                             preferred_element_type=jnp.float32
                             ).astype(jnp.bfloat16)
            acc = (b[Cout:2 * Cout, :].astype(jnp.float32)
                   + pltpu.roll(b[0:Cout, :], plane, 1).astype(jnp.float32)
                   * mask_ref[17:18, :].astype(jnp.float32)
                   + pltpu.roll(b[2 * Cout:3 * Cout, :], V - plane, 1
                                ).astype(jnp.float32)
                   * mask_ref[18:19, :].astype(jnp.float32))
            alpha = alpha_ref[li].astype(jnp.float32)
            return jnp.where(acc > 0, acc, alpha * acc)

        for pb in range(2):
            src_ref[pb] = x_ref[pb].astype(jnp.bfloat16)
        for pb in range(2):
            act_ref[pb] = conv_bn_prelu(src_ref[pb], w1_ref, 0,
                                        pb).astype(jnp.bfloat16)
        for pb in range(2):
            act_ref[pb] = conv_bn_prelu(act_ref[pb], w2_ref, 1,
                                        pb).astype(jnp.bfloat16)
        for pb in range(2):
            o_ref[pb] = conv_bn_prelu(act_ref[pb], w3_ref, 2, pb)

    return _kernel_body


def _pack_weight(w, b, scale, shift, cout):
    cin = w.shape[3]
    w_f = w * scale[None, None, None, None, :]
    w_t = jnp.transpose(w_f, (0, 4, 1, 2, 3))
    w_m = w_t.reshape(3 * cout, 9 * cin)
    shift_p = jnp.zeros((3 * cout,), jnp.float32).at[cout:2 * cout].set(
        shift + b * scale)
    extra = jnp.concatenate(
        [shift_p[:, None], jnp.zeros((3 * cout, 7), jnp.float32)], axis=1)
    return jnp.concatenate([w_m, extra], axis=1).astype(jnp.bfloat16)


def _build_masks(D, H, W):
    V = D * H * W
    idx = jnp.arange(V, dtype=jnp.int32)
    w_i = idx % W
    h_i = (idx // W) % H
    d_i = idx // (H * W)
    rows = []
    for t in range(9):
        kh, kw = divmod(t, 3)
        ok_h = jnp.logical_and(h_i + (kh - 1) >= 0, h_i + (kh - 1) <= H - 1)
        ok_w = jnp.logical_and(w_i + (kw - 1) >= 0, w_i + (kw - 1) <= W - 1)
        rows.append(jnp.logical_and(ok_h, ok_w).astype(jnp.float32))
    rows.append(jnp.ones((V,), jnp.float32))
    for _ in range(7):
        rows.append(jnp.zeros((V,), jnp.float32))
    rows.append((d_i >= 1).astype(jnp.float32))
    rows.append((d_i <= D - 2).astype(jnp.float32))
    for _ in range(5):
        rows.append(jnp.zeros((V,), jnp.float32))
    return jnp.stack(rows, axis=0).astype(jnp.bfloat16)


def kernel(x, w1, b1, s1, t1, a1, w2, b2, s2, t2, a2, w3, b3, s3, t3, a3):
    N, Cin, D, H, W = x.shape
    Cout = w1.shape[-1]
    V = D * H * W

    x_c = x.reshape(N, Cin, V)
    masks = _build_masks(D, H, W)
    wf1 = _pack_weight(w1, b1, s1, t1, Cout)
    wf2 = _pack_weight(w2, b2, s2, t2, Cout)
    wf3 = _pack_weight(w3, b3, s3, t3, Cout)
    alphas = jnp.stack([
        jnp.broadcast_to(jnp.asarray(a, jnp.float32), (Cout,))
        for a in (a1, a2, a3)], axis=0).reshape(3, Cout, 1).astype(jnp.bfloat16)

    _fused = _make_fused_kernel(D, H, W, Cout)

    out = pl.pallas_call(
        _fused,
        out_shape=jax.ShapeDtypeStruct((N, Cout, V), jnp.float32),
        grid=(N // 2,),
        in_specs=[
            pl.BlockSpec((2, Cin, V), lambda n: (n, 0, 0)),
            pl.BlockSpec((3 * Cout, 9 * Cin + 8), lambda n: (0, 0)),
            pl.BlockSpec((3 * Cout, 9 * Cout + 8), lambda n: (0, 0)),
            pl.BlockSpec((3 * Cout, 9 * Cout + 8), lambda n: (0, 0)),
            pl.BlockSpec((3, Cout, 1), lambda n: (0, 0, 0)),
            pl.BlockSpec((24, V), lambda n: (0, 0)),
        ],
        out_specs=pl.BlockSpec((2, Cout, V), lambda n: (n, 0, 0)),
        scratch_shapes=[
            pltpu.VMEM((2, Cin, V), jnp.bfloat16),
            pltpu.VMEM((2, Cout, V), jnp.bfloat16),
            pltpu.VMEM((2, 9 * Cout + 8, V), jnp.bfloat16),
            pltpu.VMEM((2, 3 * Cout, V), jnp.bfloat16),
        ],
        compiler_params=pltpu.CompilerParams(
            dimension_semantics=("parallel",)),
    )(x_c, wf1, wf2, wf3, alphas, masks)

    return out.reshape(N, Cout, D, H, W)


# speedup vs baseline: 3.9419x; 1.0328x over previous
import jax
import jax.numpy as jnp
from jax.experimental import pallas as pl
from jax.experimental.pallas import tpu as pltpu


def _make_fused_kernel(D, H, W, Cout):
    V = D * H * W
    plane = H * W

    def _kernel_body(x_ref, w1_ref, w2_ref, w3_ref, p_ref, mask_ref,
                     o_ref, src_ref, act_ref, patch_ref, b_ref):

        def conv_bn_prelu(src, w_ref, li, pb):
            C = src.shape[0]
            patch = patch_ref.at[pb]
            b = b_ref.at[pb]
            for t in range(9):
                kh, kw = divmod(t, 3)
                s = (kh - 1) * W + (kw - 1)
                sh = (-s) % V
                piece = src if sh == 0 else pltpu.roll(src, sh, 1)
                patch[t * C:(t + 1) * C, :] = piece * mask_ref[t:t + 1, :]
            b[...] = jnp.dot(w_ref[...], patch[:9 * C, :],
                             preferred_element_type=jnp.float32)
            acc = (b[Cout:2 * Cout, :]
                   + pltpu.roll(b[0:Cout, :], plane, 1)
                   * mask_ref[17:18, :].astype(jnp.float32)
                   + pltpu.roll(b[2 * Cout:3 * Cout, :], V - plane, 1)
                   * mask_ref[18:19, :].astype(jnp.float32))
            scale, shift, alpha = p_ref[li, 0], p_ref[li, 1], p_ref[li, 2]
            y = acc * scale + shift
            return jnp.where(y > 0, y, alpha * y)

        for pb in range(2):
            src_ref[pb] = x_ref[pb].astype(jnp.bfloat16)
        for pb in range(2):
            act_ref[pb] = conv_bn_prelu(src_ref[pb], w1_ref, 0,
                                        pb).astype(jnp.bfloat16)
        for pb in range(2):
            act_ref[pb] = conv_bn_prelu(act_ref[pb], w2_ref, 1,
                                        pb).astype(jnp.bfloat16)
        for pb in range(2):
            o_ref[pb] = conv_bn_prelu(act_ref[pb], w3_ref, 2, pb)

    return _kernel_body


def _pack_weight(w, cout):
    cin = w.shape[3]
    w_t = jnp.transpose(w, (0, 4, 1, 2, 3))
    return w_t.reshape(3 * cout, 9 * cin).astype(jnp.bfloat16)


def _pack_params(b, scale, shift, alpha, cout):
    alpha_p = jnp.broadcast_to(jnp.asarray(alpha, jnp.float32), (cout,))
    return jnp.stack([scale, shift + b * scale, alpha_p], axis=0)


def _build_masks(D, H, W):
    V = D * H * W
    idx = jnp.arange(V, dtype=jnp.int32)
    w_i = idx % W
    h_i = (idx // W) % H
    d_i = idx // (H * W)
    rows = []
    for t in range(9):
        kh, kw = divmod(t, 3)
        ok_h = jnp.logical_and(h_i + (kh - 1) >= 0, h_i + (kh - 1) <= H - 1)
        ok_w = jnp.logical_and(w_i + (kw - 1) >= 0, w_i + (kw - 1) <= W - 1)
        rows.append(jnp.logical_and(ok_h, ok_w).astype(jnp.float32))
    for _ in range(8):
        rows.append(jnp.zeros((V,), jnp.float32))
    rows.append((d_i >= 1).astype(jnp.float32))
    rows.append((d_i <= D - 2).astype(jnp.float32))
    for _ in range(5):
        rows.append(jnp.zeros((V,), jnp.float32))
    return jnp.stack(rows, axis=0).astype(jnp.bfloat16)


def kernel(x, w1, b1, s1, t1, a1, w2, b2, s2, t2, a2, w3, b3, s3, t3, a3):
    N, Cin, D, H, W = x.shape
    Cout = w1.shape[-1]
    V = D * H * W

    x_c = x.reshape(N, Cin, V)
    masks = _build_masks(D, H, W)
    wf1 = _pack_weight(w1, Cout)
    wf2 = _pack_weight(w2, Cout)
    wf3 = _pack_weight(w3, Cout)
    prm = jnp.stack([_pack_params(b1, s1, t1, a1, Cout),
                     _pack_params(b2, s2, t2, a2, Cout),
                     _pack_params(b3, s3, t3, a3, Cout)],
                    axis=0).reshape(3, 3, Cout, 1)

    _fused = _make_fused_kernel(D, H, W, Cout)

    out = pl.pallas_call(
        _fused,
        out_shape=jax.ShapeDtypeStruct((N, Cout, V), jnp.float32),
        grid=(N // 2,),
        in_specs=[
            pl.BlockSpec((2, Cin, V), lambda n: (n, 0, 0)),
            pl.BlockSpec((3 * Cout, 9 * Cin), lambda n: (0, 0)),
            pl.BlockSpec((3 * Cout, 9 * Cout), lambda n: (0, 0)),
            pl.BlockSpec((3 * Cout, 9 * Cout), lambda n: (0, 0)),
            pl.BlockSpec((3, 3, Cout, 1), lambda n: (0, 0, 0, 0)),
            pl.BlockSpec((24, V), lambda n: (0, 0)),
        ],
        out_specs=pl.BlockSpec((2, Cout, V), lambda n: (n, 0, 0)),
        scratch_shapes=[
            pltpu.VMEM((2, Cin, V), jnp.bfloat16),
            pltpu.VMEM((2, Cout, V), jnp.bfloat16),
            pltpu.VMEM((2, 9 * Cout, V), jnp.bfloat16),
            pltpu.VMEM((2, 3 * Cout, V), jnp.float32),
        ],
        compiler_params=pltpu.CompilerParams(
            dimension_semantics=("parallel",)),
    )(x_c, wf1, wf2, wf3, prm, masks)

    return out.reshape(N, Cout, D, H, W)


# speedup vs baseline: 3.9931x; 1.0130x over previous
import jax
import jax.numpy as jnp
from jax.experimental import pallas as pl
from jax.experimental.pallas import tpu as pltpu


def _make_fused_kernel(D, H, W, Cout):
    V = D * H * W
    plane = H * W

    def _kernel_body(x_ref, w1_ref, w2_ref, w3_ref, p_ref, mask_ref,
                     o_ref, src_ref, act_ref, patch_ref, b_ref):

        def conv_bn_prelu(src, w_ref, li, pb):
            C = src.shape[0]
            patch = patch_ref.at[pb % 2]
            b = b_ref.at[pb % 2]
            for t in range(9):
                kh, kw = divmod(t, 3)
                s = (kh - 1) * W + (kw - 1)
                sh = (-s) % V
                piece = src if sh == 0 else pltpu.roll(src, sh, 1)
                patch[t * C:(t + 1) * C, :] = piece * mask_ref[t:t + 1, :]
            b[...] = jnp.dot(w_ref[...], patch[:9 * C, :],
                             preferred_element_type=jnp.float32)
            acc = (b[Cout:2 * Cout, :]
                   + pltpu.roll(b[0:Cout, :], plane, 1)
                   * mask_ref[17:18, :].astype(jnp.float32)
                   + pltpu.roll(b[2 * Cout:3 * Cout, :], V - plane, 1)
                   * mask_ref[18:19, :].astype(jnp.float32))
            scale, shift, alpha = p_ref[li, 0], p_ref[li, 1], p_ref[li, 2]
            y = acc * scale + shift
            return jnp.where(y > 0, y, alpha * y)

        for pb in range(4):
            src_ref[pb] = x_ref[pb].astype(jnp.bfloat16)
        for pb in range(4):
            act_ref[pb] = conv_bn_prelu(src_ref[pb], w1_ref, 0,
                                        pb).astype(jnp.bfloat16)
        for pb in range(4):
            act_ref[pb] = conv_bn_prelu(act_ref[pb], w2_ref, 1,
                                        pb).astype(jnp.bfloat16)
        for pb in range(4):
            o_ref[pb] = conv_bn_prelu(act_ref[pb], w3_ref, 2, pb)

    return _kernel_body


def _pack_weight(w, cout):
    cin = w.shape[3]
    w_t = jnp.transpose(w, (0, 4, 1, 2, 3))
    return w_t.reshape(3 * cout, 9 * cin).astype(jnp.bfloat16)


def _pack_params(b, scale, shift, alpha, cout):
    alpha_p = jnp.broadcast_to(jnp.asarray(alpha, jnp.float32), (cout,))
    return jnp.stack([scale, shift + b * scale, alpha_p], axis=0)


def _build_masks(D, H, W):
    V = D * H * W
    idx = jnp.arange(V, dtype=jnp.int32)
    w_i = idx % W
    h_i = (idx // W) % H
    d_i = idx // (H * W)
    rows = []
    for t in range(9):
        kh, kw = divmod(t, 3)
        ok_h = jnp.logical_and(h_i + (kh - 1) >= 0, h_i + (kh - 1) <= H - 1)
        ok_w = jnp.logical_and(w_i + (kw - 1) >= 0, w_i + (kw - 1) <= W - 1)
        rows.append(jnp.logical_and(ok_h, ok_w).astype(jnp.float32))
    for _ in range(8):
        rows.append(jnp.zeros((V,), jnp.float32))
    rows.append((d_i >= 1).astype(jnp.float32))
    rows.append((d_i <= D - 2).astype(jnp.float32))
    for _ in range(5):
        rows.append(jnp.zeros((V,), jnp.float32))
    return jnp.stack(rows, axis=0).astype(jnp.bfloat16)


def kernel(x, w1, b1, s1, t1, a1, w2, b2, s2, t2, a2, w3, b3, s3, t3, a3):
    N, Cin, D, H, W = x.shape
    Cout = w1.shape[-1]
    V = D * H * W

    x_c = x.reshape(N, Cin, V)
    masks = _build_masks(D, H, W)
    wf1 = _pack_weight(w1, Cout)
    wf2 = _pack_weight(w2, Cout)
    wf3 = _pack_weight(w3, Cout)
    prm = jnp.stack([_pack_params(b1, s1, t1, a1, Cout),
                     _pack_params(b2, s2, t2, a2, Cout),
                     _pack_params(b3, s3, t3, a3, Cout)],
                    axis=0).reshape(3, 3, Cout, 1)

    _fused = _make_fused_kernel(D, H, W, Cout)

    out = pl.pallas_call(
        _fused,
        out_shape=jax.ShapeDtypeStruct((N, Cout, V), jnp.float32),
        grid=(N // 4,),
        in_specs=[
            pl.BlockSpec((4, Cin, V), lambda n: (n, 0, 0)),
            pl.BlockSpec((3 * Cout, 9 * Cin), lambda n: (0, 0)),
            pl.BlockSpec((3 * Cout, 9 * Cout), lambda n: (0, 0)),
            pl.BlockSpec((3 * Cout, 9 * Cout), lambda n: (0, 0)),
            pl.BlockSpec((3, 3, Cout, 1), lambda n: (0, 0, 0, 0)),
            pl.BlockSpec((24, V), lambda n: (0, 0)),
        ],
        out_specs=pl.BlockSpec((4, Cout, V), lambda n: (n, 0, 0)),
        scratch_shapes=[
            pltpu.VMEM((4, Cin, V), jnp.bfloat16),
            pltpu.VMEM((4, Cout, V), jnp.bfloat16),
            pltpu.VMEM((2, 9 * Cout, V), jnp.bfloat16),
            pltpu.VMEM((2, 3 * Cout, V), jnp.float32),
        ],
        compiler_params=pltpu.CompilerParams(
            dimension_semantics=("parallel",)),
    )(x_c, wf1, wf2, wf3, prm, masks)

    return out.reshape(N, Cout, D, H, W)
